# Initial kernel scaffold; baseline (speedup 1.0000x reference)
import jax
import jax.numpy as jnp
from jax.experimental import pallas as pl


def kernel(x, edge_index, batch, num_graphs, Wg0, bg0, Wg1, bg1, Wg2, bg2, Wc1, bc1, Wc2, bc2, Wn1, bn1, Wn2, bn2):
    raise NotImplementedError("write your pallas kernel here")


# speedup vs baseline: 13.3646x; 13.3646x over previous
import functools

import jax
import jax.numpy as jnp
from jax import lax
from jax.experimental import pallas as pl
from jax.experimental.pallas import tpu as pltpu
from jax.experimental.pallas import tpu_sc as plsc

N = 10000
NP = 10240
E = 320000
D = 128
G = 64
NC, NS = 2, 16
NW = NC * NS
EPT = E // NW
CH = 125
NCHUNK = EPT // CH
GRP = 16
RPT = NP // NS
NB = NP // 128

_sc_mesh = plsc.VectorSubcoreMesh(
    core_axis_name="c", subcore_axis_name="s", num_cores=NC, num_subcores=NS)


@functools.partial(
    pl.kernel,
    out_type=jax.ShapeDtypeStruct((NC, NP, 8), jnp.float32),
    mesh=_sc_mesh,
    scratch_types=[
        pltpu.VMEM((NCHUNK, CH), jnp.int32),
        pltpu.VMEM((CH, 8), jnp.float32),
        pltpu.VMEM_SHARED((NP, 8), jnp.float32),
    ],
)
def _sc_degree(dst_hbm, ones_hbm, zeros_hbm, out_hbm,
               dst_v, ones_v, deg_sp):
    c = lax.axis_index("c")
    s = lax.axis_index("s")
    wid = c * NS + s
    base = s * RPT
    pltpu.sync_copy(zeros_hbm, deg_sp.at[pl.ds(base, RPT)])
    pltpu.sync_copy(ones_hbm, ones_v)
    pltpu.sync_copy(dst_hbm.at[wid], dst_v)
    plsc.subcore_barrier()

    def body(j, carry):
        pltpu.sync_copy(ones_v, deg_sp.at[dst_v.at[j]], add=True)
        return carry
    lax.fori_loop(0, NCHUNK, body, 0)
    plsc.subcore_barrier()
    pltpu.sync_copy(deg_sp.at[pl.ds(base, RPT)], out_hbm.at[c, pl.ds(base, RPT)])


@functools.partial(
    pl.kernel,
    out_type=jax.ShapeDtypeStruct((NC, NP, D), jnp.float32),
    mesh=_sc_mesh,
    scratch_types=[
        pltpu.VMEM((GRP, CH), jnp.int32),
        pltpu.VMEM((GRP, CH), jnp.int32),
        pltpu.VMEM((CH, D), jnp.float32),
        pltpu.VMEM((CH, D), jnp.float32),
        pltpu.SemaphoreType.DMA,
        pltpu.SemaphoreType.DMA,
        pltpu.VMEM_SHARED((NP, D), jnp.float32),
    ],
)
def _sc_scatter(hp_hbm, src_hbm, dst_hbm, zeros_hbm, out_hbm,
                src_v, dst_v, rows0, rows1, sem0, sem1, s_sp):
    c = lax.axis_index("c")
    s = lax.axis_index("s")
    wid = c * NS + s
    base = s * RPT
    pltpu.sync_copy(zeros_hbm, s_sp.at[pl.ds(base, RPT)])
    plsc.subcore_barrier()

    def group(g, carry):
        pltpu.sync_copy(src_hbm.at[wid, pl.ds(g * GRP, GRP)], src_v)
        pltpu.sync_copy(dst_hbm.at[wid, pl.ds(g * GRP, GRP)], dst_v)

        def body(jj, carry2):
            j0 = 2 * jj
            j1 = 2 * jj + 1
            cp0 = pltpu.async_copy(hp_hbm.at[src_v.at[j0]], rows0, sem0)
            cp1 = pltpu.async_copy(hp_hbm.at[src_v.at[j1]], rows1, sem1)
            cp0.wait()
            pltpu.sync_copy(rows0, s_sp.at[dst_v.at[j0]], add=True)
            cp1.wait()
            pltpu.sync_copy(rows1, s_sp.at[dst_v.at[j1]], add=True)
            return carry2
        lax.fori_loop(0, GRP // 2, body, 0)
        return carry
    lax.fori_loop(0, NCHUNK // GRP, group, 0)
    plsc.subcore_barrier()
    pltpu.sync_copy(s_sp.at[pl.ds(base, RPT)], out_hbm.at[c, pl.ds(base, RPT)])


def _tc_prep_body(x_ref, deg_ref, out_ref):
    dv = deg_ref[...]
    inv = lax.rsqrt(dv[0] + dv[1] + 1.0)
    out_ref[...] = x_ref[...] * inv


_tc_prep = pl.pallas_call(
    _tc_prep_body,
    grid=(NB,),
    in_specs=[
        pl.BlockSpec((128, D), lambda b: (b, 0)),
        pl.BlockSpec((NC, 128, 1), lambda b: (0, b, 0)),
    ],
    out_specs=pl.BlockSpec((128, D), lambda b: (b, 0)),
    out_shape=jax.ShapeDtypeStruct((NP, D), jnp.float32),
)


def _tc_layer_body(p_ref, hp_ref, deg_ref, w_ref, b_ref, out_ref):
    dv = deg_ref[...]
    inv = lax.rsqrt(dv[0] + dv[1] + 1.0)
    pv = p_ref[...]
    agg = (pv[0] + pv[1] + hp_ref[...]) * inv
    h = jnp.maximum(
        jnp.dot(agg, w_ref[...], preferred_element_type=jnp.float32)
        + b_ref[...], 0.0)
    out_ref[...] = h * inv


_tc_layer = pl.pallas_call(
    _tc_layer_body,
    grid=(NB,),
    in_specs=[
        pl.BlockSpec((NC, 128, D), lambda b: (0, b, 0)),
        pl.BlockSpec((128, D), lambda b: (b, 0)),
        pl.BlockSpec((NC, 128, 1), lambda b: (0, b, 0)),
        pl.BlockSpec((D, D), lambda b: (0, 0)),
        pl.BlockSpec((1, D), lambda b: (0, 0)),
    ],
    out_specs=pl.BlockSpec((128, D), lambda b: (b, 0)),
    out_shape=jax.ShapeDtypeStruct((NP, D), jnp.float32),
)


def _tc_final_body(p_ref, hp_ref, deg_ref, w_ref, b_ref, batch_ref,
                   wc1_ref, bc1_ref, wc2_ref, bc2_ref,
                   wn1_ref, bn1_ref, wn2_ref, bn2_ref,
                   fc_ref, fn_ref, y_ref, pool_acc, cnt_acc):
    blk = pl.program_id(0)

    @pl.when(blk == 0)
    def _():
        pool_acc[...] = jnp.zeros_like(pool_acc)
        cnt_acc[...] = jnp.zeros_like(cnt_acc)

    dv = deg_ref[...]
    inv = lax.rsqrt(dv[0] + dv[1] + 1.0)
    pv = p_ref[...]
    agg = (pv[0] + pv[1] + hp_ref[...]) * inv
    h = jnp.maximum(
        jnp.dot(agg, w_ref[...], preferred_element_type=jnp.float32)
        + b_ref[...], 0.0)
    bv = batch_ref[...].reshape(1, 128)
    gid = lax.broadcasted_iota(jnp.int32, (G, 128), 0)
    onehot = (gid == bv).astype(jnp.float32)
    pool_acc[...] += jnp.dot(onehot, h, preferred_element_type=jnp.float32)
    cnt_acc[...] += jnp.sum(onehot, axis=1, keepdims=True)

    @pl.when(blk == NB - 1)
    def _():
        y = pool_acc[...] / jnp.maximum(cnt_acc[...], 1.0)
        hc = jnp.maximum(
            jnp.dot(y, wc1_ref[...], preferred_element_type=jnp.float32)
            + bc1_ref[...], 0.0)
        fc_ref[...] = (
            jnp.dot(hc, wc2_ref[...], preferred_element_type=jnp.float32)
            + bc2_ref[...])
        hn = jnp.maximum(
            jnp.dot(y, wn1_ref[...], preferred_element_type=jnp.float32)
            + bn1_ref[...], 0.0)
        fn_ref[...] = (
            jnp.dot(hn, wn2_ref[...], preferred_element_type=jnp.float32)
            + bn2_ref[...])
        y_ref[...] = y


_vec_spec = pl.BlockSpec((1, D), lambda b: (0, 0))
_mat_spec = pl.BlockSpec((D, D), lambda b: (0, 0))

_tc_final = pl.pallas_call(
    _tc_final_body,
    grid=(NB,),
    in_specs=[
        pl.BlockSpec((NC, 128, D), lambda b: (0, b, 0)),
        pl.BlockSpec((128, D), lambda b: (b, 0)),
        pl.BlockSpec((NC, 128, 1), lambda b: (0, b, 0)),
        _mat_spec, _vec_spec,
        pl.BlockSpec((1, 1, 128), lambda b: (b, 0, 0)),
        _mat_spec, _vec_spec, _mat_spec, _vec_spec,
        _mat_spec, _vec_spec, _mat_spec, _vec_spec,
    ],
    out_specs=[
        pl.BlockSpec((G, D), lambda b: (0, 0)),
        pl.BlockSpec((G, D), lambda b: (0, 0)),
        pl.BlockSpec((G, D), lambda b: (0, 0)),
    ],
    out_shape=[
        jax.ShapeDtypeStruct((G, D), jnp.float32),
        jax.ShapeDtypeStruct((G, D), jnp.float32),
        jax.ShapeDtypeStruct((G, D), jnp.float32),
    ],
    scratch_shapes=[
        pltpu.VMEM((G, D), jnp.float32),
        pltpu.VMEM((G, 1), jnp.float32),
    ],
)


def kernel(x, edge_index, batch, num_graphs,
           Wg0, bg0, Wg1, bg1, Wg2, bg2,
           Wc1, bc1, Wc2, bc2, Wn1, bn1, Wn2, bn2):
    del num_graphs
    x_pad = jnp.pad(x, ((0, NP - N), (0, 0)))
    src_r = edge_index[0].reshape(NW, NCHUNK, CH)
    dst_r = edge_index[1].reshape(NW, NCHUNK, CH)
    batch3 = jnp.pad(batch, (0, NP - N), constant_values=G).reshape(NB, 1, 128)
    zeros8 = jnp.zeros((RPT, 8), jnp.float32)
    ones8 = jnp.ones((CH, 8), jnp.float32)
    zeros_d = jnp.zeros((RPT, D), jnp.float32)

    degp = _sc_degree(dst_r, ones8, zeros8)[:, :, 0:1]
    h0p = _tc_prep(x_pad, degp)
    p = _sc_scatter(h0p, src_r, dst_r, zeros_d)
    h1p = _tc_layer(p, h0p, degp, Wg0, bg0.reshape(1, D))
    p = _sc_scatter(h1p, src_r, dst_r, zeros_d)
    h2p = _tc_layer(p, h1p, degp, Wg1, bg1.reshape(1, D))
    p = _sc_scatter(h2p, src_r, dst_r, zeros_d)
    fc, fn, y = _tc_final(
        p, h2p, degp, Wg2, bg2.reshape(1, D), batch3,
        Wc1, bc1.reshape(1, D), Wc2, bc2.reshape(1, D),
        Wn1, bn1.reshape(1, D), Wn2, bn2.reshape(1, D))
    return (fc, fn, y)


# speedup vs baseline: 15.5394x; 1.1627x over previous
import functools

import jax
import jax.numpy as jnp
from jax import lax
from jax.experimental import pallas as pl
from jax.experimental.pallas import tpu as pltpu
from jax.experimental.pallas import tpu_sc as plsc

N = 10000
NP = 10240
E = 320000
D = 128
G = 64
NC, NS = 2, 16
NW = NC * NS
EPT = E // NW
CH = 125
NCHUNK = EPT // CH
GRP = 16
RPT = NP // NS
NB = NP // 128

_sc_mesh = plsc.VectorSubcoreMesh(
    core_axis_name="c", subcore_axis_name="s", num_cores=NC, num_subcores=NS)


@functools.partial(
    pl.kernel,
    out_type=jax.ShapeDtypeStruct((NC, NP, 8), jnp.float32),
    mesh=_sc_mesh,
    scratch_types=[
        pltpu.VMEM((NCHUNK, CH), jnp.int32),
        pltpu.VMEM((CH, 8), jnp.float32),
        pltpu.VMEM_SHARED((NP, 8), jnp.float32),
    ],
)
def _sc_degree(dst_hbm, ones_hbm, zeros_hbm, out_hbm,
               dst_v, ones_v, deg_sp):
    c = lax.axis_index("c")
    s = lax.axis_index("s")
    wid = c * NS + s
    base = s * RPT
    pltpu.sync_copy(zeros_hbm, deg_sp.at[pl.ds(base, RPT)])
    pltpu.sync_copy(ones_hbm, ones_v)
    pltpu.sync_copy(dst_hbm.at[wid], dst_v)
    plsc.subcore_barrier()

    def body(j, carry):
        pltpu.sync_copy(ones_v, deg_sp.at[dst_v.at[j]], add=True)
        return carry
    lax.fori_loop(0, NCHUNK, body, 0)
    plsc.subcore_barrier()
    pltpu.sync_copy(deg_sp.at[pl.ds(base, RPT)], out_hbm.at[c, pl.ds(base, RPT)])


@functools.partial(
    pl.kernel,
    out_type=jax.ShapeDtypeStruct((NC, NP, D), jnp.float32),
    mesh=_sc_mesh,
    scratch_types=[
        pltpu.VMEM((GRP, CH), jnp.int32),
        pltpu.VMEM((GRP, CH), jnp.int32),
        pltpu.VMEM((CH, D), jnp.float32),
        pltpu.VMEM((CH, D), jnp.float32),
        pltpu.SemaphoreType.DMA,
        pltpu.SemaphoreType.DMA,
        pltpu.SemaphoreType.DMA,
        pltpu.VMEM_SHARED((NP, D), jnp.float32),
    ],
)
def _sc_scatter(hp_hbm, src_hbm, dst_hbm, zeros_hbm, out_hbm,
                src_v, dst_v, rows0, rows1, gsem0, gsem1, ssem, s_sp):
    c = lax.axis_index("c")
    s = lax.axis_index("s")
    wid = c * NS + s
    base = s * RPT
    pltpu.sync_copy(zeros_hbm, s_sp.at[pl.ds(base, RPT)])
    plsc.subcore_barrier()

    def wait_gather(j, buf, sem):
        pltpu.make_async_copy(hp_hbm.at[src_v.at[j]], buf, sem).wait()

    def wait_scatter(j, buf, sem):
        pltpu.make_async_copy(buf, s_sp.at[dst_v.at[j]], sem).wait()

    def group(g, carry):
        pltpu.sync_copy(src_hbm.at[wid, pl.ds(g * GRP, GRP)], src_v)
        pltpu.sync_copy(dst_hbm.at[wid, pl.ds(g * GRP, GRP)], dst_v)
        pltpu.async_copy(hp_hbm.at[src_v.at[0]], rows0, gsem0)
        pltpu.async_copy(hp_hbm.at[src_v.at[1]], rows1, gsem1)

        def pair(jj, carry2):
            j0 = 2 * jj
            j1 = j0 + 1
            wait_gather(j0, rows0, gsem0)
            pltpu.async_copy(rows0, s_sp.at[dst_v.at[j0]], ssem, add=True)
            wait_gather(j1, rows1, gsem1)
            wait_scatter(j0, rows0, ssem)
            pltpu.async_copy(rows1, s_sp.at[dst_v.at[j1]], ssem, add=True)
            pltpu.async_copy(hp_hbm.at[src_v.at[j0 + 2]], rows0, gsem0)
            wait_scatter(j1, rows1, ssem)
            pltpu.async_copy(hp_hbm.at[src_v.at[j1 + 2]], rows1, gsem1)
            return carry2
        lax.fori_loop(0, GRP // 2 - 1, pair, 0)
        wait_gather(GRP - 2, rows0, gsem0)
        pltpu.async_copy(rows0, s_sp.at[dst_v.at[GRP - 2]], ssem, add=True)
        wait_gather(GRP - 1, rows1, gsem1)
        wait_scatter(GRP - 2, rows0, ssem)
        pltpu.async_copy(rows1, s_sp.at[dst_v.at[GRP - 1]], ssem, add=True)
        wait_scatter(GRP - 1, rows1, ssem)
        return carry
    lax.fori_loop(0, NCHUNK // GRP, group, 0)
    plsc.subcore_barrier()
    pltpu.sync_copy(s_sp.at[pl.ds(base, RPT)], out_hbm.at[c, pl.ds(base, RPT)])


def _tc_prep_body(x_ref, deg_ref, out_ref):
    dv = deg_ref[...]
    inv = lax.rsqrt(dv[0] + dv[1] + 1.0)
    out_ref[...] = x_ref[...] * inv


_tc_prep = pl.pallas_call(
    _tc_prep_body,
    grid=(NB,),
    in_specs=[
        pl.BlockSpec((128, D), lambda b: (b, 0)),
        pl.BlockSpec((NC, 128, 1), lambda b: (0, b, 0)),
    ],
    out_specs=pl.BlockSpec((128, D), lambda b: (b, 0)),
    out_shape=jax.ShapeDtypeStruct((NP, D), jnp.float32),
)


def _tc_layer_body(p_ref, hp_ref, deg_ref, w_ref, b_ref, out_ref):
    dv = deg_ref[...]
    inv = lax.rsqrt(dv[0] + dv[1] + 1.0)
    pv = p_ref[...]
    agg = (pv[0] + pv[1] + hp_ref[...]) * inv
    h = jnp.maximum(
        jnp.dot(agg, w_ref[...], preferred_element_type=jnp.float32)
        + b_ref[...], 0.0)
    out_ref[...] = h * inv


_tc_layer = pl.pallas_call(
    _tc_layer_body,
    grid=(NB,),
    in_specs=[
        pl.BlockSpec((NC, 128, D), lambda b: (0, b, 0)),
        pl.BlockSpec((128, D), lambda b: (b, 0)),
        pl.BlockSpec((NC, 128, 1), lambda b: (0, b, 0)),
        pl.BlockSpec((D, D), lambda b: (0, 0)),
        pl.BlockSpec((1, D), lambda b: (0, 0)),
    ],
    out_specs=pl.BlockSpec((128, D), lambda b: (b, 0)),
    out_shape=jax.ShapeDtypeStruct((NP, D), jnp.float32),
)


def _tc_final_body(p_ref, hp_ref, deg_ref, w_ref, b_ref, batch_ref,
                   wc1_ref, bc1_ref, wc2_ref, bc2_ref,
                   wn1_ref, bn1_ref, wn2_ref, bn2_ref,
                   fc_ref, fn_ref, y_ref, pool_acc, cnt_acc):
    blk = pl.program_id(0)

    @pl.when(blk == 0)
    def _():
        pool_acc[...] = jnp.zeros_like(pool_acc)
        cnt_acc[...] = jnp.zeros_like(cnt_acc)

    dv = deg_ref[...]
    inv = lax.rsqrt(dv[0] + dv[1] + 1.0)
    pv = p_ref[...]
    agg = (pv[0] + pv[1] + hp_ref[...]) * inv
    h = jnp.maximum(
        jnp.dot(agg, w_ref[...], preferred_element_type=jnp.float32)
        + b_ref[...], 0.0)
    bv = batch_ref[...].reshape(1, 128)
    gid = lax.broadcasted_iota(jnp.int32, (G, 128), 0)
    onehot = (gid == bv).astype(jnp.float32)
    pool_acc[...] += jnp.dot(onehot, h, preferred_element_type=jnp.float32)
    cnt_acc[...] += jnp.sum(onehot, axis=1, keepdims=True)

    @pl.when(blk == NB - 1)
    def _():
        y = pool_acc[...] / jnp.maximum(cnt_acc[...], 1.0)
        hc = jnp.maximum(
            jnp.dot(y, wc1_ref[...], preferred_element_type=jnp.float32)
            + bc1_ref[...], 0.0)
        fc_ref[...] = (
            jnp.dot(hc, wc2_ref[...], preferred_element_type=jnp.float32)
            + bc2_ref[...])
        hn = jnp.maximum(
            jnp.dot(y, wn1_ref[...], preferred_element_type=jnp.float32)
            + bn1_ref[...], 0.0)
        fn_ref[...] = (
            jnp.dot(hn, wn2_ref[...], preferred_element_type=jnp.float32)
            + bn2_ref[...])
        y_ref[...] = y


_vec_spec = pl.BlockSpec((1, D), lambda b: (0, 0))
_mat_spec = pl.BlockSpec((D, D), lambda b: (0, 0))

_tc_final = pl.pallas_call(
    _tc_final_body,
    grid=(NB,),
    in_specs=[
        pl.BlockSpec((NC, 128, D), lambda b: (0, b, 0)),
        pl.BlockSpec((128, D), lambda b: (b, 0)),
        pl.BlockSpec((NC, 128, 1), lambda b: (0, b, 0)),
        _mat_spec, _vec_spec,
        pl.BlockSpec((1, 1, 128), lambda b: (b, 0, 0)),
        _mat_spec, _vec_spec, _mat_spec, _vec_spec,
        _mat_spec, _vec_spec, _mat_spec, _vec_spec,
    ],
    out_specs=[
        pl.BlockSpec((G, D), lambda b: (0, 0)),
        pl.BlockSpec((G, D), lambda b: (0, 0)),
        pl.BlockSpec((G, D), lambda b: (0, 0)),
    ],
    out_shape=[
        jax.ShapeDtypeStruct((G, D), jnp.float32),
        jax.ShapeDtypeStruct((G, D), jnp.float32),
        jax.ShapeDtypeStruct((G, D), jnp.float32),
    ],
    scratch_shapes=[
        pltpu.VMEM((G, D), jnp.float32),
        pltpu.VMEM((G, 1), jnp.float32),
    ],
)


def kernel(x, edge_index, batch, num_graphs,
           Wg0, bg0, Wg1, bg1, Wg2, bg2,
           Wc1, bc1, Wc2, bc2, Wn1, bn1, Wn2, bn2):
    del num_graphs
    x_pad = jnp.pad(x, ((0, NP - N), (0, 0)))
    src_r = edge_index[0].reshape(NW, NCHUNK, CH)
    dst_r = edge_index[1].reshape(NW, NCHUNK, CH)
    batch3 = jnp.pad(batch, (0, NP - N), constant_values=G).reshape(NB, 1, 128)
    zeros8 = jnp.zeros((RPT, 8), jnp.float32)
    ones8 = jnp.ones((CH, 8), jnp.float32)
    zeros_d = jnp.zeros((RPT, D), jnp.float32)

    degp = _sc_degree(dst_r, ones8, zeros8)[:, :, 0:1]
    h0p = _tc_prep(x_pad, degp)
    p = _sc_scatter(h0p, src_r, dst_r, zeros_d)
    h1p = _tc_layer(p, h0p, degp, Wg0, bg0.reshape(1, D))
    p = _sc_scatter(h1p, src_r, dst_r, zeros_d)
    h2p = _tc_layer(p, h1p, degp, Wg1, bg1.reshape(1, D))
    p = _sc_scatter(h2p, src_r, dst_r, zeros_d)
    fc, fn, y = _tc_final(
        p, h2p, degp, Wg2, bg2.reshape(1, D), batch3,
        Wc1, bc1.reshape(1, D), Wc2, bc2.reshape(1, D),
        Wn1, bn1.reshape(1, D), Wn2, bn2.reshape(1, D))
    return (fc, fn, y)


# speedup vs baseline: 16.0652x; 1.0338x over previous
import functools

import jax
import jax.numpy as jnp
from jax import lax
from jax.experimental import pallas as pl
from jax.experimental.pallas import tpu as pltpu
from jax.experimental.pallas import tpu_sc as plsc

N = 10000
NP = 10240
E = 320000
D = 128
G = 64
NC, NS = 2, 16
NW = NC * NS
EPT = E // NW
CH = 125
NCHUNK = EPT // CH
GRP = 40
NG = NCHUNK // GRP
RPT = NP // NS
NB = NP // 128

_sc_mesh = plsc.VectorSubcoreMesh(
    core_axis_name="c", subcore_axis_name="s", num_cores=NC, num_subcores=NS)


@functools.partial(
    pl.kernel,
    out_type=jax.ShapeDtypeStruct((NC, NP, 8), jnp.float32),
    mesh=_sc_mesh,
    scratch_types=[
        pltpu.VMEM((NCHUNK, CH), jnp.int32),
        pltpu.VMEM((CH, 8), jnp.float32),
        pltpu.VMEM_SHARED((NP, 8), jnp.float32),
    ],
)
def _sc_degree(dst_hbm, ones_hbm, zeros_hbm, out_hbm,
               dst_v, ones_v, deg_sp):
    c = lax.axis_index("c")
    s = lax.axis_index("s")
    wid = c * NS + s
    base = s * RPT
    pltpu.sync_copy(zeros_hbm, deg_sp.at[pl.ds(base, RPT)])
    pltpu.sync_copy(ones_hbm, ones_v)
    pltpu.sync_copy(dst_hbm.at[wid], dst_v)
    plsc.subcore_barrier()

    def body(j, carry):
        pltpu.sync_copy(ones_v, deg_sp.at[dst_v.at[j]], add=True)
        return carry
    lax.fori_loop(0, NCHUNK, body, 0)
    plsc.subcore_barrier()
    pltpu.sync_copy(deg_sp.at[pl.ds(base, RPT)], out_hbm.at[c, pl.ds(base, RPT)])


@functools.partial(
    pl.kernel,
    out_type=jax.ShapeDtypeStruct((NC, NP, D), jnp.float32),
    mesh=_sc_mesh,
    scratch_types=[
        pltpu.VMEM((GRP, CH), jnp.int32),
        pltpu.VMEM((GRP, CH), jnp.int32),
        pltpu.VMEM((CH, D), jnp.float32),
        pltpu.VMEM((CH, D), jnp.float32),
        pltpu.SemaphoreType.DMA,
        pltpu.SemaphoreType.DMA,
        pltpu.SemaphoreType.DMA,
        pltpu.VMEM_SHARED((NP, D), jnp.float32),
    ],
)
def _sc_scatter(hp_hbm, src_hbm, dst_hbm, zeros_hbm, out_hbm,
                src_v, dst_v, rows0, rows1, gsem0, gsem1, ssem, s_sp):
    c = lax.axis_index("c")
    s = lax.axis_index("s")
    wid = c * NS + s
    base = s * RPT
    pltpu.sync_copy(zeros_hbm, s_sp.at[pl.ds(base, RPT)])
    plsc.subcore_barrier()

    def wait_gather(j, buf, sem):
        pltpu.make_async_copy(hp_hbm.at[src_v.at[j]], buf, sem).wait()

    def wait_scatter(j, buf, sem):
        pltpu.make_async_copy(buf, s_sp.at[dst_v.at[j]], sem).wait()

    def group(g, carry):
        pltpu.sync_copy(src_hbm.at[wid, pl.ds(g * GRP, GRP)], src_v)
        pltpu.sync_copy(dst_hbm.at[wid, pl.ds(g * GRP, GRP)], dst_v)
        pltpu.async_copy(hp_hbm.at[src_v.at[0]], rows0, gsem0)
        pltpu.async_copy(hp_hbm.at[src_v.at[1]], rows1, gsem1)

        def pair(jj, carry2):
            j0 = 2 * jj
            j1 = j0 + 1
            wait_gather(j0, rows0, gsem0)
            pltpu.async_copy(rows0, s_sp.at[dst_v.at[j0]], ssem, add=True)
            wait_gather(j1, rows1, gsem1)
            wait_scatter(j0, rows0, ssem)
            pltpu.async_copy(rows1, s_sp.at[dst_v.at[j1]], ssem, add=True)
            pltpu.async_copy(hp_hbm.at[src_v.at[j0 + 2]], rows0, gsem0)
            wait_scatter(j1, rows1, ssem)
            pltpu.async_copy(hp_hbm.at[src_v.at[j1 + 2]], rows1, gsem1)
            return carry2
        lax.fori_loop(0, GRP // 2 - 1, pair, 0)
        wait_gather(GRP - 2, rows0, gsem0)
        pltpu.async_copy(rows0, s_sp.at[dst_v.at[GRP - 2]], ssem, add=True)
        wait_gather(GRP - 1, rows1, gsem1)
        wait_scatter(GRP - 2, rows0, ssem)
        pltpu.async_copy(rows1, s_sp.at[dst_v.at[GRP - 1]], ssem, add=True)
        wait_scatter(GRP - 1, rows1, ssem)
        return carry
    lax.fori_loop(0, NG, group, 0)
    plsc.subcore_barrier()
    pltpu.sync_copy(s_sp.at[pl.ds(base, RPT)], out_hbm.at[c, pl.ds(base, RPT)])


def _tc_prep_body(x_ref, deg_ref, out_ref):
    dv = deg_ref[...]
    inv = lax.rsqrt(dv[0] + dv[1] + 1.0)
    out_ref[...] = x_ref[...] * inv


_tc_prep = pl.pallas_call(
    _tc_prep_body,
    grid=(NB,),
    in_specs=[
        pl.BlockSpec((128, D), lambda b: (b, 0)),
        pl.BlockSpec((NC, 128, 1), lambda b: (0, b, 0)),
    ],
    out_specs=pl.BlockSpec((128, D), lambda b: (b, 0)),
    out_shape=jax.ShapeDtypeStruct((NP, D), jnp.float32),
)


def _tc_layer_body(p_ref, hp_ref, deg_ref, w_ref, b_ref, out_ref):
    dv = deg_ref[...]
    inv = lax.rsqrt(dv[0] + dv[1] + 1.0)
    pv = p_ref[...]
    agg = (pv[0] + pv[1] + hp_ref[...]) * inv
    h = jnp.maximum(
        jnp.dot(agg, w_ref[...], preferred_element_type=jnp.float32)
        + b_ref[...], 0.0)
    out_ref[...] = h * inv


_tc_layer = pl.pallas_call(
    _tc_layer_body,
    grid=(NB,),
    in_specs=[
        pl.BlockSpec((NC, 128, D), lambda b: (0, b, 0)),
        pl.BlockSpec((128, D), lambda b: (b, 0)),
        pl.BlockSpec((NC, 128, 1), lambda b: (0, b, 0)),
        pl.BlockSpec((D, D), lambda b: (0, 0)),
        pl.BlockSpec((1, D), lambda b: (0, 0)),
    ],
    out_specs=pl.BlockSpec((128, D), lambda b: (b, 0)),
    out_shape=jax.ShapeDtypeStruct((NP, D), jnp.float32),
)


def _tc_final_body(p_ref, hp_ref, deg_ref, w_ref, b_ref, batch_ref,
                   wc1_ref, bc1_ref, wc2_ref, bc2_ref,
                   wn1_ref, bn1_ref, wn2_ref, bn2_ref,
                   fc_ref, fn_ref, y_ref, pool_acc, cnt_acc):
    blk = pl.program_id(0)

    @pl.when(blk == 0)
    def _():
        pool_acc[...] = jnp.zeros_like(pool_acc)
        cnt_acc[...] = jnp.zeros_like(cnt_acc)

    dv = deg_ref[...]
    inv = lax.rsqrt(dv[0] + dv[1] + 1.0)
    pv = p_ref[...]
    agg = (pv[0] + pv[1] + hp_ref[...]) * inv
    h = jnp.maximum(
        jnp.dot(agg, w_ref[...], preferred_element_type=jnp.float32)
        + b_ref[...], 0.0)
    bv = batch_ref[...].reshape(1, 128)
    gid = lax.broadcasted_iota(jnp.int32, (G, 128), 0)
    onehot = (gid == bv).astype(jnp.float32)
    pool_acc[...] += jnp.dot(onehot, h, preferred_element_type=jnp.float32)
    cnt_acc[...] += jnp.sum(onehot, axis=1, keepdims=True)

    @pl.when(blk == NB - 1)
    def _():
        y = pool_acc[...] / jnp.maximum(cnt_acc[...], 1.0)
        hc = jnp.maximum(
            jnp.dot(y, wc1_ref[...], preferred_element_type=jnp.float32)
            + bc1_ref[...], 0.0)
        fc_ref[...] = (
            jnp.dot(hc, wc2_ref[...], preferred_element_type=jnp.float32)
            + bc2_ref[...])
        hn = jnp.maximum(
            jnp.dot(y, wn1_ref[...], preferred_element_type=jnp.float32)
            + bn1_ref[...], 0.0)
        fn_ref[...] = (
            jnp.dot(hn, wn2_ref[...], preferred_element_type=jnp.float32)
            + bn2_ref[...])
        y_ref[...] = y


_vec_spec = pl.BlockSpec((1, D), lambda b: (0, 0))
_mat_spec = pl.BlockSpec((D, D), lambda b: (0, 0))

_tc_final = pl.pallas_call(
    _tc_final_body,
    grid=(NB,),
    in_specs=[
        pl.BlockSpec((NC, 128, D), lambda b: (0, b, 0)),
        pl.BlockSpec((128, D), lambda b: (b, 0)),
        pl.BlockSpec((NC, 128, 1), lambda b: (0, b, 0)),
        _mat_spec, _vec_spec,
        pl.BlockSpec((1, 1, 128), lambda b: (b, 0, 0)),
        _mat_spec, _vec_spec, _mat_spec, _vec_spec,
        _mat_spec, _vec_spec, _mat_spec, _vec_spec,
    ],
    out_specs=[
        pl.BlockSpec((G, D), lambda b: (0, 0)),
        pl.BlockSpec((G, D), lambda b: (0, 0)),
        pl.BlockSpec((G, D), lambda b: (0, 0)),
    ],
    out_shape=[
        jax.ShapeDtypeStruct((G, D), jnp.float32),
        jax.ShapeDtypeStruct((G, D), jnp.float32),
        jax.ShapeDtypeStruct((G, D), jnp.float32),
    ],
    scratch_shapes=[
        pltpu.VMEM((G, D), jnp.float32),
        pltpu.VMEM((G, 1), jnp.float32),
    ],
)


def kernel(x, edge_index, batch, num_graphs,
           Wg0, bg0, Wg1, bg1, Wg2, bg2,
           Wc1, bc1, Wc2, bc2, Wn1, bn1, Wn2, bn2):
    del num_graphs
    x_pad = jnp.pad(x, ((0, NP - N), (0, 0)))
    src_r = edge_index[0].reshape(NW, NCHUNK, CH)
    dst_r = edge_index[1].reshape(NW, NCHUNK, CH)
    batch3 = jnp.pad(batch, (0, NP - N), constant_values=G).reshape(NB, 1, 128)
    zeros8 = jnp.zeros((RPT, 8), jnp.float32)
    ones8 = jnp.ones((CH, 8), jnp.float32)
    zeros_d = jnp.zeros((RPT, D), jnp.float32)

    degp = _sc_degree(dst_r, ones8, zeros8)[:, :, 0:1]
    h0p = _tc_prep(x_pad, degp)
    p = _sc_scatter(h0p, src_r, dst_r, zeros_d)
    h1p = _tc_layer(p, h0p, degp, Wg0, bg0.reshape(1, D))
    p = _sc_scatter(h1p, src_r, dst_r, zeros_d)
    h2p = _tc_layer(p, h1p, degp, Wg1, bg1.reshape(1, D))
    p = _sc_scatter(h2p, src_r, dst_r, zeros_d)
    fc, fn, y = _tc_final(
        p, h2p, degp, Wg2, bg2.reshape(1, D), batch3,
        Wc1, bc1.reshape(1, D), Wc2, bc2.reshape(1, D),
        Wn1, bn1.reshape(1, D), Wn2, bn2.reshape(1, D))
    return (fc, fn, y)


# speedup vs baseline: 19.6665x; 1.2242x over previous
import functools

import jax
import jax.numpy as jnp
from jax import lax
from jax.experimental import pallas as pl
from jax.experimental.pallas import tpu as pltpu
from jax.experimental.pallas import tpu_sc as plsc

N = 10000
NP = 10240
E = 320000
D = 128
G = 64
NC, NS = 2, 16
NW = NC * NS
EPT = E // NW
CH = 125
NCHUNK = EPT // CH
GRP = 40
NG = NCHUNK // GRP
RPT = NP // NS
BLK = 1024
NB = NP // BLK
FBLK = 128
NFB = NP // FBLK

_sc_mesh = plsc.VectorSubcoreMesh(
    core_axis_name="c", subcore_axis_name="s", num_cores=NC, num_subcores=NS)


@functools.partial(
    pl.kernel,
    out_type=jax.ShapeDtypeStruct((NC, NP, 8), jnp.float32),
    mesh=_sc_mesh,
    scratch_types=[
        pltpu.VMEM((NCHUNK, CH), jnp.int32),
        pltpu.VMEM((CH, 8), jnp.float32),
        pltpu.VMEM_SHARED((NP, 8), jnp.float32),
    ],
)
def _sc_degree(dst_hbm, ones_hbm, zeros_hbm, out_hbm,
               dst_v, ones_v, deg_sp):
    c = lax.axis_index("c")
    s = lax.axis_index("s")
    wid = c * NS + s
    base = s * RPT
    pltpu.sync_copy(zeros_hbm, deg_sp.at[pl.ds(base, RPT)])
    pltpu.sync_copy(ones_hbm, ones_v)
    pltpu.sync_copy(dst_hbm.at[wid], dst_v)
    plsc.subcore_barrier()

    def body(j, carry):
        pltpu.sync_copy(ones_v, deg_sp.at[dst_v.at[j]], add=True)
        return carry
    lax.fori_loop(0, NCHUNK, body, 0)
    plsc.subcore_barrier()
    pltpu.sync_copy(deg_sp.at[pl.ds(base, RPT)], out_hbm.at[c, pl.ds(base, RPT)])


@functools.partial(
    pl.kernel,
    out_type=jax.ShapeDtypeStruct((NC, NP, D), jnp.float32),
    mesh=_sc_mesh,
    scratch_types=[
        pltpu.VMEM((GRP, CH), jnp.int32),
        pltpu.VMEM((GRP, CH), jnp.int32),
        pltpu.VMEM((CH, D), jnp.float32),
        pltpu.VMEM((CH, D), jnp.float32),
        pltpu.SemaphoreType.DMA,
        pltpu.SemaphoreType.DMA,
        pltpu.SemaphoreType.DMA,
        pltpu.VMEM_SHARED((NP, D), jnp.float32),
    ],
)
def _sc_scatter(hp_hbm, src_hbm, dst_hbm, zeros_hbm, out_hbm,
                src_v, dst_v, rows0, rows1, gsem0, gsem1, ssem, s_sp):
    c = lax.axis_index("c")
    s = lax.axis_index("s")
    wid = c * NS + s
    base = s * RPT
    pltpu.sync_copy(zeros_hbm, s_sp.at[pl.ds(base, RPT)])
    plsc.subcore_barrier()

    def wait_gather(j, buf, sem):
        pltpu.make_async_copy(hp_hbm.at[src_v.at[j]], buf, sem).wait()

    def wait_scatter(j, buf, sem):
        pltpu.make_async_copy(buf, s_sp.at[dst_v.at[j]], sem).wait()

    def group(g, carry):
        pltpu.sync_copy(src_hbm.at[wid, pl.ds(g * GRP, GRP)], src_v)
        pltpu.sync_copy(dst_hbm.at[wid, pl.ds(g * GRP, GRP)], dst_v)
        pltpu.async_copy(hp_hbm.at[src_v.at[0]], rows0, gsem0)
        pltpu.async_copy(hp_hbm.at[src_v.at[1]], rows1, gsem1)

        def pair(jj, carry2):
            j0 = 2 * jj
            j1 = j0 + 1
            wait_gather(j0, rows0, gsem0)
            pltpu.async_copy(rows0, s_sp.at[dst_v.at[j0]], ssem, add=True)
            wait_gather(j1, rows1, gsem1)
            wait_scatter(j0, rows0, ssem)
            pltpu.async_copy(rows1, s_sp.at[dst_v.at[j1]], ssem, add=True)
            pltpu.async_copy(hp_hbm.at[src_v.at[j0 + 2]], rows0, gsem0)
            wait_scatter(j1, rows1, ssem)
            pltpu.async_copy(hp_hbm.at[src_v.at[j1 + 2]], rows1, gsem1)
            return carry2
        lax.fori_loop(0, GRP // 2 - 1, pair, 0)
        wait_gather(GRP - 2, rows0, gsem0)
        pltpu.async_copy(rows0, s_sp.at[dst_v.at[GRP - 2]], ssem, add=True)
        wait_gather(GRP - 1, rows1, gsem1)
        wait_scatter(GRP - 2, rows0, ssem)
        pltpu.async_copy(rows1, s_sp.at[dst_v.at[GRP - 1]], ssem, add=True)
        wait_scatter(GRP - 1, rows1, ssem)
        return carry
    lax.fori_loop(0, NG, group, 0)
    plsc.subcore_barrier()
    pltpu.sync_copy(s_sp.at[pl.ds(base, RPT)], out_hbm.at[c, pl.ds(base, RPT)])


def _tc_prep_body(x_ref, deg_ref, out_ref):
    dv = deg_ref[...]
    inv = lax.rsqrt(dv[0] + dv[1] + 1.0)
    out_ref[...] = x_ref[...] * inv


_tc_prep = pl.pallas_call(
    _tc_prep_body,
    grid=(NB,),
    in_specs=[
        pl.BlockSpec((BLK, D), lambda b: (b, 0)),
        pl.BlockSpec((NC, BLK, 1), lambda b: (0, b, 0)),
    ],
    out_specs=pl.BlockSpec((BLK, D), lambda b: (b, 0)),
    out_shape=jax.ShapeDtypeStruct((NP, D), jnp.float32),
)


def _tc_layer_body(p_ref, hp_ref, deg_ref, w_ref, b_ref, out_ref):
    dv = deg_ref[...]
    inv = lax.rsqrt(dv[0] + dv[1] + 1.0)
    pv = p_ref[...]
    agg = (pv[0] + pv[1] + hp_ref[...]) * inv
    h = jnp.maximum(
        jnp.dot(agg, w_ref[...], preferred_element_type=jnp.float32)
        + b_ref[...], 0.0)
    out_ref[...] = h * inv


_tc_layer = pl.pallas_call(
    _tc_layer_body,
    grid=(NB,),
    in_specs=[
        pl.BlockSpec((NC, BLK, D), lambda b: (0, b, 0)),
        pl.BlockSpec((BLK, D), lambda b: (b, 0)),
        pl.BlockSpec((NC, BLK, 1), lambda b: (0, b, 0)),
        pl.BlockSpec((D, D), lambda b: (0, 0)),
        pl.BlockSpec((1, D), lambda b: (0, 0)),
    ],
    out_specs=pl.BlockSpec((BLK, D), lambda b: (b, 0)),
    out_shape=jax.ShapeDtypeStruct((NP, D), jnp.float32),
)


def _tc_final_body(p_ref, hp_ref, deg_ref, w_ref, b_ref, batch_ref,
                   wc1_ref, bc1_ref, wc2_ref, bc2_ref,
                   wn1_ref, bn1_ref, wn2_ref, bn2_ref,
                   fc_ref, fn_ref, y_ref, pool_acc, cnt_acc):
    blk = pl.program_id(0)

    @pl.when(blk == 0)
    def _():
        pool_acc[...] = jnp.zeros_like(pool_acc)
        cnt_acc[...] = jnp.zeros_like(cnt_acc)

    dv = deg_ref[...]
    inv = lax.rsqrt(dv[0] + dv[1] + 1.0)
    pv = p_ref[...]
    agg = (pv[0] + pv[1] + hp_ref[...]) * inv
    h = jnp.maximum(
        jnp.dot(agg, w_ref[...], preferred_element_type=jnp.float32)
        + b_ref[...], 0.0)
    bv = batch_ref[...].reshape(1, FBLK)
    gid = lax.broadcasted_iota(jnp.int32, (G, FBLK), 0)
    onehot = (gid == bv).astype(jnp.float32)
    pool_acc[...] += jnp.dot(onehot, h, preferred_element_type=jnp.float32)
    cnt_acc[...] += jnp.sum(onehot, axis=1, keepdims=True)

    @pl.when(blk == NFB - 1)
    def _():
        y = pool_acc[...] / jnp.maximum(cnt_acc[...], 1.0)
        hc = jnp.maximum(
            jnp.dot(y, wc1_ref[...], preferred_element_type=jnp.float32)
            + bc1_ref[...], 0.0)
        fc_ref[...] = (
            jnp.dot(hc, wc2_ref[...], preferred_element_type=jnp.float32)
            + bc2_ref[...])
        hn = jnp.maximum(
            jnp.dot(y, wn1_ref[...], preferred_element_type=jnp.float32)
            + bn1_ref[...], 0.0)
        fn_ref[...] = (
            jnp.dot(hn, wn2_ref[...], preferred_element_type=jnp.float32)
            + bn2_ref[...])
        y_ref[...] = y


_vec_spec = pl.BlockSpec((1, D), lambda b: (0, 0))
_mat_spec = pl.BlockSpec((D, D), lambda b: (0, 0))

_tc_final = pl.pallas_call(
    _tc_final_body,
    grid=(NFB,),
    in_specs=[
        pl.BlockSpec((NC, FBLK, D), lambda b: (0, b, 0)),
        pl.BlockSpec((FBLK, D), lambda b: (b, 0)),
        pl.BlockSpec((NC, FBLK, 1), lambda b: (0, b, 0)),
        _mat_spec, _vec_spec,
        pl.BlockSpec((1, 1, FBLK), lambda b: (b, 0, 0)),
        _mat_spec, _vec_spec, _mat_spec, _vec_spec,
        _mat_spec, _vec_spec, _mat_spec, _vec_spec,
    ],
    out_specs=[
        pl.BlockSpec((G, D), lambda b: (0, 0)),
        pl.BlockSpec((G, D), lambda b: (0, 0)),
        pl.BlockSpec((G, D), lambda b: (0, 0)),
    ],
    out_shape=[
        jax.ShapeDtypeStruct((G, D), jnp.float32),
        jax.ShapeDtypeStruct((G, D), jnp.float32),
        jax.ShapeDtypeStruct((G, D), jnp.float32),
    ],
    scratch_shapes=[
        pltpu.VMEM((G, D), jnp.float32),
        pltpu.VMEM((G, 1), jnp.float32),
    ],
)


def kernel(x, edge_index, batch, num_graphs,
           Wg0, bg0, Wg1, bg1, Wg2, bg2,
           Wc1, bc1, Wc2, bc2, Wn1, bn1, Wn2, bn2):
    del num_graphs
    x_pad = jnp.pad(x, ((0, NP - N), (0, 0)))
    src_r = edge_index[0].reshape(NW, NCHUNK, CH)
    dst_r = edge_index[1].reshape(NW, NCHUNK, CH)
    batch3 = jnp.pad(batch, (0, NP - N), constant_values=G).reshape(NFB, 1, FBLK)
    zeros8 = jnp.zeros((RPT, 8), jnp.float32)
    ones8 = jnp.ones((CH, 8), jnp.float32)
    zeros_d = jnp.zeros((RPT, D), jnp.float32)

    degp = _sc_degree(dst_r, ones8, zeros8)[:, :, 0:1]
    h0p = _tc_prep(x_pad, degp)
    p = _sc_scatter(h0p, src_r, dst_r, zeros_d)
    h1p = _tc_layer(p, h0p, degp, Wg0, bg0.reshape(1, D))
    p = _sc_scatter(h1p, src_r, dst_r, zeros_d)
    h2p = _tc_layer(p, h1p, degp, Wg1, bg1.reshape(1, D))
    p = _sc_scatter(h2p, src_r, dst_r, zeros_d)
    fc, fn, y = _tc_final(
        p, h2p, degp, Wg2, bg2.reshape(1, D), batch3,
        Wc1, bc1.reshape(1, D), Wc2, bc2.reshape(1, D),
        Wn1, bn1.reshape(1, D), Wn2, bn2.reshape(1, D))
    return (fc, fn, y)


# speedup vs baseline: 21.1060x; 1.0732x over previous
import functools

import jax
import jax.numpy as jnp
from jax import lax
from jax.experimental import pallas as pl
from jax.experimental.pallas import tpu as pltpu
from jax.experimental.pallas import tpu_sc as plsc

N = 10000
NP = 10240
E = 320000
D = 128
G = 64
NC, NS = 2, 16
NW = NC * NS
EPT = E // NW
CH = 125
NCHUNK = EPT // CH
GRP = 40
NG = NCHUNK // GRP
RPT = NP // NS
BLK = 1024
NB = NP // BLK
FBLK = 512
NFB = NP // FBLK

_sc_mesh = plsc.VectorSubcoreMesh(
    core_axis_name="c", subcore_axis_name="s", num_cores=NC, num_subcores=NS)


@functools.partial(
    pl.kernel,
    out_type=jax.ShapeDtypeStruct((NC, NP, 8), jnp.float32),
    mesh=_sc_mesh,
    scratch_types=[
        pltpu.VMEM((NCHUNK, CH), jnp.int32),
        pltpu.VMEM((CH, 8), jnp.float32),
        pltpu.VMEM_SHARED((NP, 8), jnp.float32),
    ],
)
def _sc_degree(dst_hbm, ones_hbm, zeros_hbm, out_hbm,
               dst_v, ones_v, deg_sp):
    c = lax.axis_index("c")
    s = lax.axis_index("s")
    wid = c * NS + s
    base = s * RPT
    pltpu.sync_copy(zeros_hbm, deg_sp.at[pl.ds(base, RPT)])
    pltpu.sync_copy(ones_hbm, ones_v)
    pltpu.sync_copy(dst_hbm.at[wid], dst_v)
    plsc.subcore_barrier()

    def body(j, carry):
        pltpu.sync_copy(ones_v, deg_sp.at[dst_v.at[j]], add=True)
        return carry
    lax.fori_loop(0, NCHUNK, body, 0)
    plsc.subcore_barrier()
    pltpu.sync_copy(deg_sp.at[pl.ds(base, RPT)], out_hbm.at[c, pl.ds(base, RPT)])


@functools.partial(
    pl.kernel,
    out_type=jax.ShapeDtypeStruct((NC, NP, D), jnp.float32),
    mesh=_sc_mesh,
    scratch_types=[
        pltpu.VMEM((GRP, CH), jnp.int32),
        pltpu.VMEM((GRP, CH), jnp.int32),
        pltpu.VMEM((CH, D), jnp.float32),
        pltpu.VMEM((CH, D), jnp.float32),
        pltpu.SemaphoreType.DMA,
        pltpu.SemaphoreType.DMA,
        pltpu.SemaphoreType.DMA,
        pltpu.VMEM_SHARED((NP, D), jnp.float32),
    ],
)
def _sc_scatter(hp_hbm, src_hbm, dst_hbm, zeros_hbm, out_hbm,
                src_v, dst_v, rows0, rows1, gsem0, gsem1, ssem, s_sp):
    c = lax.axis_index("c")
    s = lax.axis_index("s")
    wid = c * NS + s
    base = s * RPT
    pltpu.sync_copy(zeros_hbm, s_sp.at[pl.ds(base, RPT)])
    plsc.subcore_barrier()

    def wait_gather(j, buf, sem):
        pltpu.make_async_copy(hp_hbm.at[src_v.at[j]], buf, sem).wait()

    def wait_scatter(j, buf, sem):
        pltpu.make_async_copy(buf, s_sp.at[dst_v.at[j]], sem).wait()

    def group(g, carry):
        pltpu.sync_copy(src_hbm.at[wid, pl.ds(g * GRP, GRP)], src_v)
        pltpu.sync_copy(dst_hbm.at[wid, pl.ds(g * GRP, GRP)], dst_v)
        pltpu.async_copy(hp_hbm.at[src_v.at[0]], rows0, gsem0)
        pltpu.async_copy(hp_hbm.at[src_v.at[1]], rows1, gsem1)

        def pair(jj, carry2):
            j0 = 2 * jj
            j1 = j0 + 1
            wait_gather(j0, rows0, gsem0)
            pltpu.async_copy(rows0, s_sp.at[dst_v.at[j0]], ssem, add=True)
            wait_gather(j1, rows1, gsem1)
            wait_scatter(j0, rows0, ssem)
            pltpu.async_copy(rows1, s_sp.at[dst_v.at[j1]], ssem, add=True)
            pltpu.async_copy(hp_hbm.at[src_v.at[j0 + 2]], rows0, gsem0)
            wait_scatter(j1, rows1, ssem)
            pltpu.async_copy(hp_hbm.at[src_v.at[j1 + 2]], rows1, gsem1)
            return carry2
        lax.fori_loop(0, GRP // 2 - 1, pair, 0)
        wait_gather(GRP - 2, rows0, gsem0)
        pltpu.async_copy(rows0, s_sp.at[dst_v.at[GRP - 2]], ssem, add=True)
        wait_gather(GRP - 1, rows1, gsem1)
        wait_scatter(GRP - 2, rows0, ssem)
        pltpu.async_copy(rows1, s_sp.at[dst_v.at[GRP - 1]], ssem, add=True)
        wait_scatter(GRP - 1, rows1, ssem)
        return carry
    lax.fori_loop(0, NG, group, 0)
    plsc.subcore_barrier()
    pltpu.sync_copy(s_sp.at[pl.ds(base, RPT)], out_hbm.at[c, pl.ds(base, RPT)])


def _tc_prep_body(x_ref, deg_ref, out_ref):
    dv = deg_ref[...]
    inv = lax.rsqrt(dv[0] + dv[1] + 1.0)
    out_ref[...] = x_ref[...] * inv


_tc_prep = pl.pallas_call(
    _tc_prep_body,
    grid=(NB,),
    in_specs=[
        pl.BlockSpec((BLK, D), lambda b: (b, 0)),
        pl.BlockSpec((NC, BLK, 1), lambda b: (0, b, 0)),
    ],
    out_specs=pl.BlockSpec((BLK, D), lambda b: (b, 0)),
    out_shape=jax.ShapeDtypeStruct((NP, D), jnp.float32),
)


def _tc_layer_body(p_ref, hp_ref, deg_ref, w_ref, b_ref, out_ref):
    dv = deg_ref[...]
    inv = lax.rsqrt(dv[0] + dv[1] + 1.0)
    pv = p_ref[...]
    agg = (pv[0] + pv[1] + hp_ref[...]) * inv
    h = jnp.maximum(
        jnp.dot(agg, w_ref[...], preferred_element_type=jnp.float32)
        + b_ref[...], 0.0)
    out_ref[...] = h * inv


_tc_layer = pl.pallas_call(
    _tc_layer_body,
    grid=(NB,),
    in_specs=[
        pl.BlockSpec((NC, BLK, D), lambda b: (0, b, 0)),
        pl.BlockSpec((BLK, D), lambda b: (b, 0)),
        pl.BlockSpec((NC, BLK, 1), lambda b: (0, b, 0)),
        pl.BlockSpec((D, D), lambda b: (0, 0)),
        pl.BlockSpec((1, D), lambda b: (0, 0)),
    ],
    out_specs=pl.BlockSpec((BLK, D), lambda b: (b, 0)),
    out_shape=jax.ShapeDtypeStruct((NP, D), jnp.float32),
)


def _tc_final_body(p_ref, hp_ref, deg_ref, w_ref, b_ref, oh_ref, cnt_ref,
                   wc1_ref, bc1_ref, wc2_ref, bc2_ref,
                   wn1_ref, bn1_ref, wn2_ref, bn2_ref,
                   fc_ref, fn_ref, y_ref, pool_acc):
    blk = pl.program_id(0)

    @pl.when(blk == 0)
    def _():
        pool_acc[...] = jnp.zeros_like(pool_acc)

    dv = deg_ref[...]
    inv = lax.rsqrt(dv[0] + dv[1] + 1.0)
    pv = p_ref[...]
    agg = (pv[0] + pv[1] + hp_ref[...]) * inv
    h = jnp.maximum(
        jnp.dot(agg, w_ref[...], preferred_element_type=jnp.float32)
        + b_ref[...], 0.0)
    onehot = oh_ref[...]
    pool = jnp.zeros((G, D), jnp.float32)
    for k in range(FBLK // 128):
        pool += jnp.dot(onehot[:, k * 128:(k + 1) * 128],
                        h[k * 128:(k + 1) * 128, :],
                        preferred_element_type=jnp.float32)
    pool_acc[...] += pool

    @pl.when(blk == NFB - 1)
    def _():
        y = pool_acc[...] / jnp.maximum(cnt_ref[...], 1.0)
        hc = jnp.maximum(
            jnp.dot(y, wc1_ref[...], preferred_element_type=jnp.float32)
            + bc1_ref[...], 0.0)
        fc_ref[...] = (
            jnp.dot(hc, wc2_ref[...], preferred_element_type=jnp.float32)
            + bc2_ref[...])
        hn = jnp.maximum(
            jnp.dot(y, wn1_ref[...], preferred_element_type=jnp.float32)
            + bn1_ref[...], 0.0)
        fn_ref[...] = (
            jnp.dot(hn, wn2_ref[...], preferred_element_type=jnp.float32)
            + bn2_ref[...])
        y_ref[...] = y


_vec_spec = pl.BlockSpec((1, D), lambda b: (0, 0))
_mat_spec = pl.BlockSpec((D, D), lambda b: (0, 0))

_tc_final = pl.pallas_call(
    _tc_final_body,
    grid=(NFB,),
    in_specs=[
        pl.BlockSpec((NC, FBLK, D), lambda b: (0, b, 0)),
        pl.BlockSpec((FBLK, D), lambda b: (b, 0)),
        pl.BlockSpec((NC, FBLK, 1), lambda b: (0, b, 0)),
        _mat_spec, _vec_spec,
        pl.BlockSpec((G, FBLK), lambda b: (0, b)),
        pl.BlockSpec((G, 1), lambda b: (0, 0)),
        _mat_spec, _vec_spec, _mat_spec, _vec_spec,
        _mat_spec, _vec_spec, _mat_spec, _vec_spec,
    ],
    out_specs=[
        pl.BlockSpec((G, D), lambda b: (0, 0)),
        pl.BlockSpec((G, D), lambda b: (0, 0)),
        pl.BlockSpec((G, D), lambda b: (0, 0)),
    ],
    out_shape=[
        jax.ShapeDtypeStruct((G, D), jnp.float32),
        jax.ShapeDtypeStruct((G, D), jnp.float32),
        jax.ShapeDtypeStruct((G, D), jnp.float32),
    ],
    scratch_shapes=[
        pltpu.VMEM((G, D), jnp.float32),
    ],
)


def kernel(x, edge_index, batch, num_graphs,
           Wg0, bg0, Wg1, bg1, Wg2, bg2,
           Wc1, bc1, Wc2, bc2, Wn1, bn1, Wn2, bn2):
    del num_graphs
    x_pad = jnp.pad(x, ((0, NP - N), (0, 0)))
    src_r = edge_index[0].reshape(NW, NCHUNK, CH)
    dst_r = edge_index[1].reshape(NW, NCHUNK, CH)
    batch_pad = jnp.pad(batch, (0, NP - N), constant_values=G)
    onehot_t = jax.nn.one_hot(batch_pad, G, dtype=jnp.float32).T
    counts = jnp.sum(onehot_t, axis=1, keepdims=True)
    zeros8 = jnp.zeros((RPT, 8), jnp.float32)
    ones8 = jnp.ones((CH, 8), jnp.float32)
    zeros_d = jnp.zeros((RPT, D), jnp.float32)

    degp = _sc_degree(dst_r, ones8, zeros8)[:, :, 0:1]
    h0p = _tc_prep(x_pad, degp)
    p = _sc_scatter(h0p, src_r, dst_r, zeros_d)
    h1p = _tc_layer(p, h0p, degp, Wg0, bg0.reshape(1, D))
    p = _sc_scatter(h1p, src_r, dst_r, zeros_d)
    h2p = _tc_layer(p, h1p, degp, Wg1, bg1.reshape(1, D))
    p = _sc_scatter(h2p, src_r, dst_r, zeros_d)
    fc, fn, y = _tc_final(
        p, h2p, degp, Wg2, bg2.reshape(1, D), onehot_t, counts,
        Wc1, bc1.reshape(1, D), Wc2, bc2.reshape(1, D),
        Wn1, bn1.reshape(1, D), Wn2, bn2.reshape(1, D))
    return (fc, fn, y)
